# baseline (speedup 1.0000x reference)
import functools

import jax
import jax.numpy as jnp
import numpy as np
from jax import lax
from jax.experimental import pallas as pl
from jax.experimental.pallas import tpu as pltpu
from jax.experimental.pallas import tpu_sc as plsc

B = 4096
C = 10000

MARGIN2 = 0.5
SCALE = 64.0
COS_M2 = float(np.cos(MARGIN2))
SIN_M2 = float(np.sin(MARGIN2))
THRESHOLD = float(np.cos(np.pi - MARGIN2))
THETA_MIN = -2.0

CS = 5000
TAIL = C - CS

NC = 2
NS = 16
NW = NC * NS
COLS_PER_W = B // NW
CC = 200
NCHUNK = CS // CC
NG = COLS_PER_W // 16
NRING = 4

_mesh = plsc.VectorSubcoreMesh(core_axis_name="c", subcore_axis_name="s")


@functools.partial(
    pl.kernel,
    out_type=jax.ShapeDtypeStruct((B,), jnp.float32),
    mesh=_mesh,
    scratch_types=[
        [pltpu.VMEM((CC, COLS_PER_W), jnp.float32)] * 4,
        [pltpu.VMEM((8, COLS_PER_W), jnp.float32)] * NRING,
        pltpu.VMEM((COLS_PER_W,), jnp.float32),
        pltpu.VMEM((COLS_PER_W,), jnp.float32),
        pltpu.SemaphoreType.DMA,
        pltpu.SemaphoreType.DMA,
        pltpu.SemaphoreType.DMA,
        pltpu.SemaphoreType.DMA,
        [pltpu.SemaphoreType.DMA] * NRING,
    ],
)
def _sc_label_gather(y_hbm, x_hbm, out_hbm,
                     sbufs, tilebufs, lblbuf, vv, sem0, sem1, sem2, sem3, tsems):
    wid = lax.axis_index("s") * NC + lax.axis_index("c")
    j0 = wid * COLS_PER_W

    NBUF = 4
    ssems = (sem0, sem1, sem2, sem3)
    copies = [None] * NBUF

    def scan_issue(k):
        copies[k % NBUF] = pltpu.async_copy(
            y_hbm.at[pl.ds(k * CC, CC), pl.ds(j0, COLS_PER_W)],
            sbufs[k % NBUF], ssems[k % NBUF])

    for k in range(min(NBUF, NCHUNK)):
        scan_issue(k)

    def chunk_sum(buf, c0, accs):
        def body(c, a):
            w = (c0 + 1 + c).astype(jnp.float32)
            return tuple(a[g] + buf[c, pl.ds(g * 16, 16)] * w
                         for g in range(NG))
        return plsc.parallel_loop(0, CC, 1, unroll=8, carry=tuple(accs))(body)

    accs = tuple(jnp.zeros((16,), jnp.float32) for _ in range(NG))
    for k in range(NCHUNK):
        copies[k % NBUF].wait()
        if k + NBUF < NCHUNK:
            scan_issue(k + NBUF)
        accs = chunk_sum(sbufs[k % NBUF], k * CC, accs)

    lanes = lax.iota(jnp.int32, 16)
    row8 = [None] * COLS_PER_W
    found = [None] * COLS_PER_W
    tcopies = [None] * NRING

    def issue(jj):
        lp1 = accs[jj // 16][jj % 16]
        l = jnp.maximum(lp1.astype(jnp.int32) - 1, 0)
        r8 = pl.multiple_of((l >> 3) << 3, 8)
        row8[jj] = l & 7
        found[jj] = lp1 > 0.5
        tcopies[jj % NRING] = pltpu.async_copy(
            x_hbm.at[pl.ds(r8, 8), pl.ds(j0, COLS_PER_W)],
            tilebufs[jj % NRING], tsems[jj % NRING])

    for jj in range(min(NRING, COLS_PER_W)):
        issue(jj)
    vsel = [jnp.zeros((16,), jnp.float32) for _ in range(NG)]
    for jj in range(COLS_PER_W):
        tcopies[jj % NRING].wait()
        g = jj // 16
        rv = tilebufs[jj % NRING][row8[jj], pl.ds(g * 16, 16)]
        val = rv[jj % 16]
        if jj + NRING < COLS_PER_W:
            issue(jj + NRING)
        val = jnp.where(found[jj], val, 0.0)
        vsel[g] = jnp.where(lanes == (jj % 16), val, vsel[g])
    for g in range(NG):
        vv[pl.ds(g * 16, 16)] = vsel[g]
    pltpu.sync_copy(vv, out_hbm.at[pl.ds(j0, COLS_PER_W)])


COLS_TC = 256
NCB = B // COLS_TC


def _tc_main(y_ref, x_ref, m_ref, s_ref, vt_ref):
    x = x_ref[...]
    m = jnp.max(x, axis=0)
    s = jnp.sum(jnp.exp((x - m[None, :]) * SCALE), axis=0)
    vt = jnp.sum(y_ref[...] * x[CS:, :], axis=0)
    m_ref[...] = m.reshape(1, -1)
    s_ref[...] = s.reshape(1, -1)
    vt_ref[...] = vt.reshape(1, -1)


def _combine(m_ref, s_ref, vt_ref, vs_ref, out_ref):
    m = m_ref[...]
    s = s_ref[...]
    v = vt_ref[...] + vs_ref[...]
    theta = v * COS_M2 - jnp.sqrt(jnp.maximum(1.0 - v * v, 0.0)) * SIN_M2
    w = jnp.where(v > THRESHOLD, theta, THETA_MIN - theta)
    zv = jnp.exp((v - m) * SCALE)
    zw = jnp.exp((w - m) * SCALE)
    loss = -((w - m) * SCALE - jnp.log(s - zv + zw))
    out_ref[...] = (jnp.sum(loss) * (1.0 / B)).reshape(1, 1)


@jax.jit
def kernel(y_true, norm_logits):
    yt = y_true.T
    xt = norm_logits.T

    vs = _sc_label_gather(yt, xt)

    m, s, vt = pl.pallas_call(
        _tc_main,
        grid=(NCB,),
        in_specs=[
            pl.BlockSpec((TAIL, COLS_TC), lambda i: (CS // TAIL, i)),
            pl.BlockSpec((C, COLS_TC), lambda i: (0, i)),
        ],
        out_specs=[
            pl.BlockSpec((1, COLS_TC), lambda i: (0, i)),
            pl.BlockSpec((1, COLS_TC), lambda i: (0, i)),
            pl.BlockSpec((1, COLS_TC), lambda i: (0, i)),
        ],
        out_shape=[jax.ShapeDtypeStruct((1, B), jnp.float32)] * 3,
    )(yt, xt)

    out = pl.pallas_call(
        _combine,
        out_specs=pl.BlockSpec((1, 1), lambda: (0, 0)),
        out_shape=jax.ShapeDtypeStruct((1, 1), jnp.float32),
    )(m, s, vt, vs.reshape(1, B))
    return out[0, 0]


# speedup vs baseline: 1.2791x; 1.2791x over previous
import jax
import jax.numpy as jnp
import numpy as np
from jax.experimental import pallas as pl

B = 4096
C = 10000

MARGIN2 = 0.5
SCALE = 64.0
COS_M2 = float(np.cos(MARGIN2))
SIN_M2 = float(np.sin(MARGIN2))
THRESHOLD = float(np.cos(np.pi - MARGIN2))
THETA_MIN = -2.0

COLS_PER_BLOCK = 256
NUM_BLOCKS = B // COLS_PER_BLOCK


def _arcface_block_kernel(y_ref, x_ref, out_ref):
    i = pl.program_id(0)

    x = x_ref[...]
    y = y_ref[...]

    m = jnp.max(x, axis=0)
    v = jnp.sum(y * x, axis=0)
    s = jnp.sum(jnp.exp((x - m[None, :]) * SCALE), axis=0)

    theta = v * COS_M2 - jnp.sqrt(jnp.maximum(1.0 - v * v, 0.0)) * SIN_M2
    w = jnp.where(v > THRESHOLD, theta, THETA_MIN - theta)
    zv = jnp.exp((v - m) * SCALE)
    zw = jnp.exp((w - m) * SCALE)
    s1 = s - zv + zw
    loss = -((w - m) * SCALE - jnp.log(s1))

    part = (jnp.sum(loss) * (1.0 / B)).reshape(1, 1)

    @pl.when(i == 0)
    def _():
        out_ref[...] = part

    @pl.when(i != 0)
    def _():
        out_ref[...] += part


@jax.jit
def kernel(y_true, norm_logits):
    yt = y_true.T
    xt = norm_logits.T
    out = pl.pallas_call(
        _arcface_block_kernel,
        grid=(NUM_BLOCKS,),
        in_specs=[
            pl.BlockSpec((C, COLS_PER_BLOCK), lambda i: (0, i)),
            pl.BlockSpec((C, COLS_PER_BLOCK), lambda i: (0, i)),
        ],
        out_specs=pl.BlockSpec((1, 1), lambda i: (0, 0)),
        out_shape=jax.ShapeDtypeStruct((1, 1), jnp.float32),
    )(yt, xt)
    return out[0, 0]
